# baseline (speedup 1.0000x reference)
import functools

import jax
import jax.numpy as jnp
from jax import lax
from jax.experimental import pallas as pl
from jax.experimental.pallas import tpu as pltpu
from jax.experimental.pallas import tpu_sc as plsc

N = 10000
H = 128
E = 320000
G = 64
OUT = 6

NC = 2
NS = 16
NW = NC * NS
GSZ = 128
NG = 80
HG = 40
EPAD = NW * NG * GSZ
NP = 10240
RPT = NP // NS


def _segsum_body(y_hbm, src_hbm, dst_hbm, zero_hbm, out_hbm, src_v, dst_v,
                 buf0, agg_sh, sem0):
    cid = lax.axis_index("c")
    sid = lax.axis_index("s")
    wid = cid * NS + sid
    rows = pl.ds(sid * RPT, RPT)

    pltpu.sync_copy(zero_hbm.at[rows], agg_sh.at[rows])
    pltpu.sync_copy(src_hbm.at[wid], src_v)
    pltpu.sync_copy(dst_hbm.at[wid], dst_v)
    plsc.subcore_barrier()

    def _group(g, c):
        pltpu.async_copy(y_hbm.at[src_v.at[g]], buf0, sem0).wait()
        pltpu.sync_copy(buf0, agg_sh.at[dst_v.at[g]], add=True)
        return c

    lax.fori_loop(0, NG, _group, 0)
    plsc.subcore_barrier()

    pltpu.sync_copy(agg_sh.at[rows], out_hbm.at[cid, rows])


def _segsum(y, src_g, dst_g, zeros_np):
    mesh = plsc.VectorSubcoreMesh(core_axis_name="c", subcore_axis_name="s")
    f = functools.partial(
        pl.kernel,
        mesh=mesh,
        out_type=jax.ShapeDtypeStruct((NC, NP, H), jnp.float32),
        scratch_types=[
            pltpu.VMEM((NG, GSZ), jnp.int32),
            pltpu.VMEM((NG, GSZ), jnp.int32),
            pltpu.VMEM((GSZ, H), jnp.float32),
            pltpu.VMEM_SHARED((NP, H), jnp.float32),
            pltpu.SemaphoreType.DMA,
        ],
    )(_segsum_body)
    return f(y, src_g, dst_g, zeros_np)


def _expand_body(x_ref, w_ref, o_ref):
    o_ref[...] = x_ref[...] * w_ref[...]


def _bn_relu(h, g_ref, be_ref):
    mean = jnp.mean(h, axis=0, keepdims=True)
    var = jnp.mean((h - mean) ** 2, axis=0, keepdims=True)
    h = (h - mean) / jnp.sqrt(var + 1e-5) * g_ref[...] + be_ref[...]
    return jnp.maximum(h, 0.0)


def _layer1_body(y_ref, p_ref, b1_ref, w2_ref, b2_ref, g_ref, be_ref,
                 h_ref, acc_ref):
    t = y_ref[...] + p_ref[0, :N, :] + p_ref[1, :N, :] + b1_ref[...]
    h = jnp.maximum(t, 0.0)
    h = jnp.dot(h, w2_ref[...], preferred_element_type=jnp.float32) + b2_ref[...]
    h = _bn_relu(h, g_ref, be_ref)
    h_ref[...] = h
    acc_ref[...] = h


def _layer_body(x_ref, p_ref, acc_ref, w1_ref, b1_ref, w2_ref, b2_ref,
                g_ref, be_ref, h_ref, acco_ref):
    t = x_ref[...] + p_ref[0, :N, :] + p_ref[1, :N, :]
    h = jnp.dot(t, w1_ref[...], preferred_element_type=jnp.float32) + b1_ref[...]
    h = jnp.maximum(h, 0.0)
    h = jnp.dot(h, w2_ref[...], preferred_element_type=jnp.float32) + b2_ref[...]
    h = _bn_relu(h, g_ref, be_ref)
    h_ref[...] = h
    acco_ref[...] = acc_ref[...] + h


def _final_body(acc_ref, batch_ref, w1_ref, b1_ref, w2_ref, b2_ref,
                w3_ref, b3_ref, o_ref):
    seg = lax.broadcasted_iota(jnp.int32, (G, N), 0)
    onehot = (batch_ref[...] == seg).astype(jnp.float32)
    pooled = jnp.dot(onehot, acc_ref[...], preferred_element_type=jnp.float32)
    u = jnp.maximum(
        jnp.dot(pooled, w1_ref[...], preferred_element_type=jnp.float32)
        + b1_ref[...], 0.0)
    u = jnp.maximum(
        jnp.dot(u, w2_ref[...], preferred_element_type=jnp.float32)
        + b2_ref[...], 0.0)
    o_ref[...] = (jnp.dot(u, w3_ref[...], preferred_element_type=jnp.float32)
                  + b3_ref[...])


def _tc(body, out_shapes):
    return pl.pallas_call(body, out_shape=out_shapes)


def kernel(x, edge_index, batch, W1_0, b1_0, W2_0, b2_0, gamma_0, beta_0,
           W1_r, b1_r, W2_r, b2_r, gamma_r, beta_r, fc1_W, fc1_b,
           fc2_W, fc2_b, fc3_W, fc3_b):
    f32 = jnp.float32
    src = edge_index[0]
    dst = edge_index[1]
    pad = EPAD - E
    src_g = jnp.concatenate([src, jnp.zeros((pad,), jnp.int32)]).reshape(
        NW, NG, GSZ)
    dst_g = jnp.concatenate([dst, jnp.full((pad,), NP - 1, jnp.int32)]
                            ).reshape(NW, NG, GSZ)

    nh = jax.ShapeDtypeStruct((N, H), f32)

    zeros_np = jnp.zeros((NP, H), f32)
    y = _tc(_expand_body, nh)(x, W1_0)
    p = _segsum(y, src_g, dst_g, zeros_np)
    h, acc = _tc(_layer1_body, (nh, nh))(
        y, p, b1_0.reshape(1, H), W2_0, b2_0.reshape(1, H),
        gamma_0.reshape(1, H), beta_0.reshape(1, H))

    for i in range(9):
        p = _segsum(h, src_g, dst_g, zeros_np)
        h, acc = _tc(_layer_body, (nh, nh))(
            h, p, acc, W1_r[i], b1_r[i].reshape(1, H), W2_r[i],
            b2_r[i].reshape(1, H), gamma_r[i].reshape(1, H),
            beta_r[i].reshape(1, H))

    out = _tc(_final_body, jax.ShapeDtypeStruct((G, OUT), f32))(
        acc, batch.reshape(1, N), fc1_W, fc1_b.reshape(1, H),
        fc2_W, fc2_b.reshape(1, H), fc3_W, fc3_b.reshape(1, OUT))
    return out


# speedup vs baseline: 1.7332x; 1.7332x over previous
import functools

import jax
import jax.numpy as jnp
from jax import lax
from jax.experimental import pallas as pl
from jax.experimental.pallas import tpu as pltpu
from jax.experimental.pallas import tpu_sc as plsc

N = 10000
H = 128
E = 320000
G = 64
OUT = 6

NC = 2
NS = 16
NW = NC * NS
GSZ = 128
NG = 79
EPAD = NW * NG * GSZ
NP = 10240
RPT = NP // NS


def _segsum_body(y_hbm, src_hbm, dst_hbm, out_hbm, src_v, dst_v, buf,
                 agg_sh, sem):
    cid = lax.axis_index("c")
    sid = lax.axis_index("s")
    wid = cid * NS + sid

    zeros16 = jnp.zeros((16,), jnp.float32)

    def _zrow(i, c):
        def _zcol(j, c2):
            buf[i, pl.ds(j * 16, 16)] = zeros16
            return c2
        return lax.fori_loop(0, 8, _zcol, c)

    lax.fori_loop(0, 128, _zrow, 0)
    for k in range(RPT // 128):
        pltpu.sync_copy(buf, agg_sh.at[pl.ds(sid * RPT + k * 128, 128)])

    pltpu.sync_copy(src_hbm.at[wid], src_v)
    pltpu.sync_copy(dst_hbm.at[wid], dst_v)
    plsc.subcore_barrier()

    def _group(g, c):
        pltpu.async_copy(y_hbm.at[src_v.at[g]], buf, sem).wait()
        pltpu.sync_copy(buf, agg_sh.at[dst_v.at[g]], add=True)
        return c

    lax.fori_loop(0, NG, _group, 0)
    plsc.subcore_barrier()

    for k in range(RPT // 128):
        r = sid * RPT + k * 128
        pltpu.sync_copy(agg_sh.at[pl.ds(r, 128)], buf)
        pltpu.sync_copy(buf, out_hbm.at[cid, pl.ds(r, 128)])


def _segsum(y, src_g, dst_g):
    mesh = plsc.VectorSubcoreMesh(core_axis_name="c", subcore_axis_name="s")
    f = functools.partial(
        pl.kernel,
        mesh=mesh,
        out_type=jax.ShapeDtypeStruct((NC, NP, H), jnp.float32),
        scratch_types=[
            pltpu.VMEM((NG, GSZ), jnp.int32),
            pltpu.VMEM((NG, GSZ), jnp.int32),
            pltpu.VMEM((GSZ, H), jnp.float32),
            pltpu.VMEM_SHARED((NP, H), jnp.float32),
            pltpu.SemaphoreType.DMA,
        ],
    )(_segsum_body)
    return f(y, src_g, dst_g)


def _expand_body(x_ref, w_ref, o_ref):
    o_ref[...] = x_ref[...] * w_ref[...]


def _bn_relu(h, g_ref, be_ref):
    mean = jnp.mean(h, axis=0, keepdims=True)
    var = jnp.mean((h - mean) ** 2, axis=0, keepdims=True)
    h = (h - mean) / jnp.sqrt(var + 1e-5) * g_ref[...] + be_ref[...]
    return jnp.maximum(h, 0.0)


def _layer1_body(y_ref, p_ref, b1_ref, w2_ref, b2_ref, g_ref, be_ref,
                 h_ref, acc_ref):
    t = y_ref[...] + p_ref[0, :N, :] + p_ref[1, :N, :] + b1_ref[...]
    h = jnp.maximum(t, 0.0)
    h = jnp.dot(h, w2_ref[...], preferred_element_type=jnp.float32) + b2_ref[...]
    h = _bn_relu(h, g_ref, be_ref)
    h_ref[...] = h
    acc_ref[...] = h


def _layer_body(x_ref, p_ref, acc_ref, w1_ref, b1_ref, w2_ref, b2_ref,
                g_ref, be_ref, h_ref, acco_ref):
    t = x_ref[...] + p_ref[0, :N, :] + p_ref[1, :N, :]
    h = jnp.dot(t, w1_ref[...], preferred_element_type=jnp.float32) + b1_ref[...]
    h = jnp.maximum(h, 0.0)
    h = jnp.dot(h, w2_ref[...], preferred_element_type=jnp.float32) + b2_ref[...]
    h = _bn_relu(h, g_ref, be_ref)
    h_ref[...] = h
    acco_ref[...] = acc_ref[...] + h


def _final_body(acc_ref, batch_ref, w1_ref, b1_ref, w2_ref, b2_ref,
                w3_ref, b3_ref, o_ref):
    seg = lax.broadcasted_iota(jnp.int32, (G, N), 0)
    onehot = (batch_ref[...] == seg).astype(jnp.float32)
    pooled = jnp.dot(onehot, acc_ref[...], preferred_element_type=jnp.float32)
    u = jnp.maximum(
        jnp.dot(pooled, w1_ref[...], preferred_element_type=jnp.float32)
        + b1_ref[...], 0.0)
    u = jnp.maximum(
        jnp.dot(u, w2_ref[...], preferred_element_type=jnp.float32)
        + b2_ref[...], 0.0)
    o_ref[...] = (jnp.dot(u, w3_ref[...], preferred_element_type=jnp.float32)
                  + b3_ref[...])


def _tc(body, out_shapes):
    return pl.pallas_call(body, out_shape=out_shapes)


def kernel(x, edge_index, batch, W1_0, b1_0, W2_0, b2_0, gamma_0, beta_0,
           W1_r, b1_r, W2_r, b2_r, gamma_r, beta_r, fc1_W, fc1_b,
           fc2_W, fc2_b, fc3_W, fc3_b):
    f32 = jnp.float32
    src = edge_index[0]
    dst = edge_index[1]
    pad = EPAD - E
    src_g = jnp.concatenate([src, jnp.zeros((pad,), jnp.int32)]).reshape(
        NW, NG, GSZ)
    dst_g = jnp.concatenate([dst, jnp.full((pad,), NP - 1, jnp.int32)]
                            ).reshape(NW, NG, GSZ)

    nh = jax.ShapeDtypeStruct((N, H), f32)

    y = _tc(_expand_body, nh)(x, W1_0)
    p = _segsum(y, src_g, dst_g)
    h, acc = _tc(_layer1_body, (nh, nh))(
        y, p, b1_0.reshape(1, H), W2_0, b2_0.reshape(1, H),
        gamma_0.reshape(1, H), beta_0.reshape(1, H))

    for i in range(9):
        p = _segsum(h, src_g, dst_g)
        h, acc = _tc(_layer_body, (nh, nh))(
            h, p, acc, W1_r[i], b1_r[i].reshape(1, H), W2_r[i],
            b2_r[i].reshape(1, H), gamma_r[i].reshape(1, H),
            beta_r[i].reshape(1, H))

    out = _tc(_final_body, jax.ShapeDtypeStruct((G, OUT), f32))(
        acc, batch.reshape(1, N), fc1_W, fc1_b.reshape(1, H),
        fc2_W, fc2_b.reshape(1, H), fc3_W, fc3_b.reshape(1, OUT))
    return out
